# baseline (speedup 1.0000x reference)
import functools

import jax
import jax.numpy as jnp
from jax import lax
from jax.experimental import pallas as pl
from jax.experimental.pallas import tpu as pltpu
from jax.experimental.pallas import tpu_sc as plsc

VOCAB = 100000
EMBED_DIM = 128
OUT_DIM = 2
BATCH = 4096
SEQ = 200

NC, NS, L = 2, 16, 16
VB = 2048
VPAD = ((VOCAB + VB - 1) // VB) * VB
ROWS_PER_G = BATCH // NS
CHUNK = 32
NCHUNK = ROWS_PER_G // CHUNK
NJ = CHUNK // L


def _proj_body(w_ref, b_ref, emb_ref, out_ref):
    out_ref[...] = (
        lax.dot_general(
            w_ref[...], emb_ref[...],
            (((1,), (1,)), ((), ())),
            preferred_element_type=jnp.float32,
        ) + b_ref[...]
    ) * (1.0 / SEQ)


def _project_table(W, b2, emb_table):
    return pl.pallas_call(
        _proj_body,
        grid=(VPAD // VB,),
        in_specs=[
            pl.BlockSpec((OUT_DIM, EMBED_DIM), lambda i: (0, 0)),
            pl.BlockSpec((OUT_DIM, 1), lambda i: (0, 0)),
            pl.BlockSpec((VB, EMBED_DIM), lambda i: (i, 0)),
        ],
        out_specs=pl.BlockSpec((OUT_DIM, VB), lambda i: (0, i)),
        out_shape=jax.ShapeDtypeStruct((OUT_DIM, VPAD), jnp.float32),
    )(W, b2, emb_table)


def _sc_pool_body(proj_hbm, tok_hbm, out_hbm,
                  table_v, idx_v0, idx_v1, out_v, sem_t, sem_c0, sem_c1):
    comp = lax.axis_index("c")
    g = lax.axis_index("s")
    bufs = [idx_v0, idx_v1]
    sems = [sem_c0, sem_c1]

    cp_table = pltpu.async_copy(proj_hbm.at[comp], table_v, sem_t)
    cps = {
        0: pltpu.async_copy(tok_hbm.at[g, 0], bufs[0], sems[0]),
        1: pltpu.async_copy(tok_hbm.at[g, 1], bufs[1], sems[1]),
    }
    cp_table.wait()

    rows = [lax.iota(jnp.int32, L) + j * L for j in range(NJ)]
    for t in range(NCHUNK):
        cps[t].wait()
        buf = bufs[t % 2]

        def body(s, accs):
            scol = jnp.full((L,), s, jnp.int32)
            new = []
            for j in range(NJ):
                tok_j = plsc.load_gather(buf, [rows[j], scol])
                new.append(accs[j] + plsc.load_gather(table_v, [tok_j]))
            return tuple(new)

        accs = lax.fori_loop(
            0, SEQ, body, tuple(jnp.zeros((L,), jnp.float32) for _ in range(NJ))
        )
        for j in range(NJ):
            out_v[pl.ds(t * CHUNK + j * L, L)] = accs[j]
        if t + 2 < NCHUNK:
            cps[t + 2] = pltpu.async_copy(tok_hbm.at[g, t + 2], buf, sems[t % 2])

    pltpu.sync_copy(out_v, out_hbm.at[comp, pl.ds(g * ROWS_PER_G, ROWS_PER_G)])


_sc_pool = functools.partial(
    pl.kernel,
    out_type=jax.ShapeDtypeStruct((OUT_DIM, BATCH), jnp.float32),
    mesh=plsc.VectorSubcoreMesh(core_axis_name="c", subcore_axis_name="s"),
    compiler_params=pltpu.CompilerParams(needs_layout_passes=False),
    scratch_types=[
        pltpu.VMEM((VPAD,), jnp.float32),
        pltpu.VMEM((CHUNK, SEQ), jnp.int32),
        pltpu.VMEM((CHUNK, SEQ), jnp.int32),
        pltpu.VMEM((ROWS_PER_G,), jnp.float32),
        pltpu.SemaphoreType.DMA,
        pltpu.SemaphoreType.DMA,
        pltpu.SemaphoreType.DMA,
    ],
)(_sc_pool_body)


def kernel(text_token, emb_table, W, b):
    tok = text_token.astype(jnp.int32)
    proj = _project_table(W, b.reshape(OUT_DIM, 1), emb_table)
    arr = tok.reshape(NS, NCHUNK, CHUNK, SEQ)
    return _sc_pool(proj, arr).T


# speedup vs baseline: 1.1143x; 1.1143x over previous
import functools

import jax
import jax.numpy as jnp
from jax import lax
from jax.experimental import pallas as pl
from jax.experimental.pallas import tpu as pltpu
from jax.experimental.pallas import tpu_sc as plsc

VOCAB = 100000
EMBED_DIM = 128
OUT_DIM = 2
BATCH = 4096
SEQ = 200

NC, NS, L = 2, 16, 16
VB = 2048
VPAD = ((VOCAB + VB - 1) // VB) * VB
ROWS_PER_G = BATCH // NS
CHUNK = 32
NCHUNK = ROWS_PER_G // CHUNK
NJ = CHUNK // L


def _proj_body(w_ref, b_ref, emb_ref, out_ref):
    out_ref[...] = (
        lax.dot_general(
            w_ref[...], emb_ref[...],
            (((1,), (1,)), ((), ())),
            preferred_element_type=jnp.float32,
        ) + b_ref[...]
    ) * (1.0 / SEQ)


def _project_table(W, b2, emb_table):
    return pl.pallas_call(
        _proj_body,
        grid=(VPAD // VB,),
        in_specs=[
            pl.BlockSpec((OUT_DIM, EMBED_DIM), lambda i: (0, 0)),
            pl.BlockSpec((OUT_DIM, 1), lambda i: (0, 0)),
            pl.BlockSpec((VB, EMBED_DIM), lambda i: (i, 0)),
        ],
        out_specs=pl.BlockSpec((OUT_DIM, VB), lambda i: (0, i)),
        out_shape=jax.ShapeDtypeStruct((OUT_DIM, VPAD), jnp.float32),
    )(W, b2, emb_table)


def _sc_pool_body(proj_hbm, tok_hbm, out_hbm,
                  table_v, idx_v0, idx_v1, out_v, sem_t, sem_c0, sem_c1):
    comp = lax.axis_index("c")
    g = lax.axis_index("s")
    bufs = [idx_v0, idx_v1]
    sems = [sem_c0, sem_c1]

    cp_table = pltpu.async_copy(proj_hbm.at[comp], table_v, sem_t)
    cps = {
        0: pltpu.async_copy(tok_hbm.at[g, 0], bufs[0], sems[0]),
        1: pltpu.async_copy(tok_hbm.at[g, 1], bufs[1], sems[1]),
    }
    cp_table.wait()

    for t in range(NCHUNK):
        cps[t].wait()
        buf = bufs[t % 2]

        def body(s, accs):
            base = s * CHUNK
            new = []
            for j in range(NJ):
                idx = buf[pl.ds(base + j * L, L)]
                new.append(accs[j] + plsc.load_gather(table_v, [idx]))
            return tuple(new)

        accs = lax.fori_loop(
            0, SEQ, body, tuple(jnp.zeros((L,), jnp.float32) for _ in range(NJ))
        )
        for j in range(NJ):
            out_v[pl.ds(t * CHUNK + j * L, L)] = accs[j]
        if t + 2 < NCHUNK:
            cps[t + 2] = pltpu.async_copy(tok_hbm.at[g, t + 2], buf, sems[t % 2])

    pltpu.sync_copy(out_v, out_hbm.at[comp, pl.ds(g * ROWS_PER_G, ROWS_PER_G)])


_sc_pool = functools.partial(
    pl.kernel,
    out_type=jax.ShapeDtypeStruct((OUT_DIM, BATCH), jnp.float32),
    mesh=plsc.VectorSubcoreMesh(core_axis_name="c", subcore_axis_name="s"),
    compiler_params=pltpu.CompilerParams(needs_layout_passes=False),
    scratch_types=[
        pltpu.VMEM((VPAD,), jnp.float32),
        pltpu.VMEM((SEQ * CHUNK,), jnp.int32),
        pltpu.VMEM((SEQ * CHUNK,), jnp.int32),
        pltpu.VMEM((ROWS_PER_G,), jnp.float32),
        pltpu.SemaphoreType.DMA,
        pltpu.SemaphoreType.DMA,
        pltpu.SemaphoreType.DMA,
    ],
)(_sc_pool_body)


def kernel(text_token, emb_table, W, b):
    tok = text_token.astype(jnp.int32)
    proj = _project_table(W, b.reshape(OUT_DIM, 1), emb_table)
    arr = (tok.reshape(NS, NCHUNK, CHUNK, SEQ)
              .transpose(0, 1, 3, 2)
              .reshape(NS, NCHUNK, SEQ * CHUNK))
    return _sc_pool(proj, arr).T


# speedup vs baseline: 1.2547x; 1.1260x over previous
import functools

import jax
import jax.numpy as jnp
from jax import lax
from jax.experimental import pallas as pl
from jax.experimental.pallas import tpu as pltpu
from jax.experimental.pallas import tpu_sc as plsc

VOCAB = 100000
EMBED_DIM = 128
OUT_DIM = 2
BATCH = 4096
SEQ = 200

NC, NS, L = 2, 16, 16
VB = 2048
VPAD = ((VOCAB + VB - 1) // VB) * VB
ROWS_PER_G = BATCH // NS
CHUNK = 64
NCHUNK = ROWS_PER_G // CHUNK
NJ = CHUNK // L


def _proj_body(w_ref, b_ref, emb_ref, out_ref):
    out_ref[...] = (
        lax.dot_general(
            w_ref[...], emb_ref[...],
            (((1,), (1,)), ((), ())),
            preferred_element_type=jnp.float32,
        ) + b_ref[...]
    ) * (1.0 / SEQ)


def _project_table(W, b2, emb_table):
    return pl.pallas_call(
        _proj_body,
        grid=(VPAD // VB,),
        in_specs=[
            pl.BlockSpec((OUT_DIM, EMBED_DIM), lambda i: (0, 0)),
            pl.BlockSpec((OUT_DIM, 1), lambda i: (0, 0)),
            pl.BlockSpec((VB, EMBED_DIM), lambda i: (i, 0)),
        ],
        out_specs=pl.BlockSpec((OUT_DIM, VB), lambda i: (0, i)),
        out_shape=jax.ShapeDtypeStruct((OUT_DIM, VPAD), jnp.float32),
    )(W, b2, emb_table)


def _sc_pool_body(proj_hbm, tok_hbm, out_hbm,
                  table_v, idx_v0, idx_v1, out_v, sem_t, sem_c0, sem_c1):
    comp = lax.axis_index("c")
    g = lax.axis_index("s")
    bufs = [idx_v0, idx_v1]
    sems = [sem_c0, sem_c1]

    cp_table = pltpu.async_copy(proj_hbm.at[comp], table_v, sem_t)
    cps = {
        0: pltpu.async_copy(tok_hbm.at[g, 0], bufs[0], sems[0]),
        1: pltpu.async_copy(tok_hbm.at[g, 1], bufs[1], sems[1]),
    }
    cp_table.wait()

    for t in range(NCHUNK):
        cps[t].wait()
        buf = bufs[t % 2]

        def body(s, accs):
            base = s * CHUNK
            new = []
            for j in range(NJ):
                idx = buf[pl.ds(base + j * L, L)]
                new.append(accs[j] + plsc.load_gather(table_v, [idx]))
            return tuple(new)

        accs = lax.fori_loop(
            0, SEQ, body, tuple(jnp.zeros((L,), jnp.float32) for _ in range(NJ)),
            unroll=2,
        )
        for j in range(NJ):
            out_v[pl.ds(t * CHUNK + j * L, L)] = accs[j]
        if t + 2 < NCHUNK:
            cps[t + 2] = pltpu.async_copy(tok_hbm.at[g, t + 2], buf, sems[t % 2])

    pltpu.sync_copy(out_v, out_hbm.at[comp, pl.ds(g * ROWS_PER_G, ROWS_PER_G)])


_sc_pool = functools.partial(
    pl.kernel,
    out_type=jax.ShapeDtypeStruct((OUT_DIM, BATCH), jnp.float32),
    mesh=plsc.VectorSubcoreMesh(core_axis_name="c", subcore_axis_name="s"),
    compiler_params=pltpu.CompilerParams(needs_layout_passes=False),
    scratch_types=[
        pltpu.VMEM((VPAD,), jnp.float32),
        pltpu.VMEM((SEQ * CHUNK,), jnp.int32),
        pltpu.VMEM((SEQ * CHUNK,), jnp.int32),
        pltpu.VMEM((ROWS_PER_G,), jnp.float32),
        pltpu.SemaphoreType.DMA,
        pltpu.SemaphoreType.DMA,
        pltpu.SemaphoreType.DMA,
    ],
)(_sc_pool_body)


def kernel(text_token, emb_table, W, b):
    tok = text_token.astype(jnp.int32)
    proj = _project_table(W, b.reshape(OUT_DIM, 1), emb_table)
    arr = (tok.reshape(NS, NCHUNK, CHUNK, SEQ)
              .transpose(0, 1, 3, 2)
              .reshape(NS, NCHUNK, SEQ * CHUNK))
    return _sc_pool(proj, arr).T
